# Initial kernel scaffold; baseline (speedup 1.0000x reference)
import jax
import jax.numpy as jnp
from jax.experimental import pallas as pl


def kernel(emissions, tags, lengths, transitions, start_transitions, end_transitions):
    raise NotImplementedError("write your pallas kernel here")


# speedup vs baseline: 49.0839x; 49.0839x over previous
import functools

import jax
import jax.numpy as jnp
from jax import lax
from jax.experimental import pallas as pl
from jax.experimental.pallas import tpu as pltpu
from jax.experimental.pallas import tpu_sc as plsc

B = 16
L = 2048
T = 16
HALF = L // 2
LN2 = 0.6931471805599453

_GDN = lax.GatherDimensionNumbers(
    offset_dims=(), collapsed_slice_dims=(0,), start_index_map=(0,))


def _bcast(vec, j):
    idx = jnp.full((T, 1), j, jnp.int32)
    return lax.gather(vec, idx, _GDN, (1,),
                      mode=lax.GatherScatterMode.PROMISE_IN_BOUNDS)


def _sc_worker(em_hbm, tags_hbm, tr_hbm, trt_hbm, st_hbm, en_hbm, pv_out, mv_out,
               em_v, tg_v, tr_v, trt_v, st_v, en_v, po_v, mo_v):
    b = lax.axis_index("s")
    h = lax.axis_index("c")
    h0 = h == 0
    iota = jnp.arange(T, dtype=jnp.int32)
    hmask = (jnp.zeros((T,), jnp.int32) + h) == 0

    row0 = b * L + h * HALF
    pltpu.sync_copy(em_hbm.at[pl.ds(row0 * T, HALF * T)],
                    em_v.at[pl.ds(0, HALF * T)])

    @pl.when(h0)
    def _():
        pltpu.sync_copy(em_hbm.at[pl.ds((row0 + HALF) * T, 8 * T)],
                        em_v.at[pl.ds(HALF * T, 8 * T)])

    tag0 = b * L + (HALF - 8) * h
    pltpu.sync_copy(tags_hbm.at[pl.ds(tag0, 1032)], tg_v.at[pl.ds(0, 1032)])
    pltpu.sync_copy(tr_hbm, tr_v)
    pltpu.sync_copy(trt_hbm, trt_v)
    pltpu.sync_copy(st_hbm, st_v)
    pltpu.sync_copy(en_hbm, en_v)

    pair_limit = jnp.where(h0, 1024, 1023)
    hoff = 8 * h

    def score_body(i, acc):
        lanes = iota + i * T
        tga = plsc.load_gather(tg_v, [lanes + hoff]) & 15
        emv = plsc.load_gather(em_v, [lanes * T + tga])
        tgb = plsc.load_gather(tg_v, [lanes + hoff + 1]) & 15
        trv = plsc.load_gather(tr_v, [tga * T + tgb])
        trv = jnp.where(lanes < pair_limit, trv, 0.0)
        return acc + emv + trv

    acc = lax.fori_loop(0, HALF // T, score_body, jnp.zeros((T,), jnp.float32))
    tE = plsc.load_gather(
        tg_v, [jnp.zeros((T,), jnp.int32) + jnp.where(h0, 0, 1031)]) & 15
    sv = plsc.load_gather(st_v, [tE])
    ev = plsc.load_gather(en_v, [tE])
    score = jnp.sum(acc) + jnp.max(jnp.where(hmask, sv, ev))

    mrows = [jnp.where(hmask,
                       jnp.exp(tr_v[pl.ds(j * T, T)]),
                       jnp.exp(trt_v[pl.ds(j * T, T)]))
             for j in range(T)]

    v0 = jnp.where(hmask, st_v[...] + em_v[pl.ds(0, T)], en_v[...])
    m0 = jnp.max(v0)
    p0 = jnp.exp(v0 - m0)

    def body(i, carry):
        p, off = carry
        row = jnp.where(h0, i + 1, 1023 - i)
        ee = jnp.exp(em_v[pl.ds(row * T, T)])
        pre = jnp.where(hmask, p, p * ee)
        a0 = _bcast(pre, 0) * mrows[0]
        a1 = _bcast(pre, 1) * mrows[1]
        a2 = _bcast(pre, 2) * mrows[2]
        a3 = _bcast(pre, 3) * mrows[3]
        for j in range(4, T, 4):
            a0 = a0 + _bcast(pre, j) * mrows[j]
            a1 = a1 + _bcast(pre, j + 1) * mrows[j + 1]
            a2 = a2 + _bcast(pre, j + 2) * mrows[j + 2]
            a3 = a3 + _bcast(pre, j + 3) * mrows[j + 3]
        pn = (a0 + a1) + (a2 + a3)
        pn = jnp.where(hmask, pn * ee, pn)
        mx = jnp.max(pn)
        ebias = (lax.bitcast_convert_type(mx, jnp.int32) >> 23) & 0xFF
        scale = lax.bitcast_convert_type((254 - ebias) << 23, jnp.float32)
        pn = pn * scale
        e_f = (ebias - 127).astype(jnp.float32)
        valid = jnp.logical_or(h0, i < 1023)
        vmask = jnp.logical_or(hmask, (jnp.zeros((T,), jnp.int32) + i) < 1023)
        p = jnp.where(vmask, pn, p)
        off = jnp.where(valid, off + e_f * LN2, off)
        return p, off

    p, off = lax.fori_loop(0, HALF, body, (p0, m0))

    po_v[...] = p
    mo_v[...] = jnp.where(iota == 0, off, jnp.where(iota == 1, score, 0.0))
    wid = h * B + b
    pltpu.sync_copy(po_v, pv_out.at[pl.ds(wid * T, T)])
    pltpu.sync_copy(mo_v, mv_out.at[pl.ds(wid * T, T)])


def _sc_crf(em_flat, tags, tr_flat, trt_flat, start, end):
    mesh = plsc.VectorSubcoreMesh(core_axis_name="c", subcore_axis_name="s")
    run = functools.partial(
        pl.kernel,
        out_type=(
            jax.ShapeDtypeStruct((2 * B * T,), jnp.float32),
            jax.ShapeDtypeStruct((2 * B * T,), jnp.float32),
        ),
        mesh=mesh,
        compiler_params=pltpu.CompilerParams(needs_layout_passes=False),
        scratch_types=[
            pltpu.VMEM(((HALF + 8) * T,), jnp.float32),
            pltpu.VMEM((1040,), jnp.int32),
            pltpu.VMEM((T * T,), jnp.float32),
            pltpu.VMEM((T * T,), jnp.float32),
            pltpu.VMEM((T,), jnp.float32),
            pltpu.VMEM((T,), jnp.float32),
            pltpu.VMEM((T,), jnp.float32),
            pltpu.VMEM((T,), jnp.float32),
        ],
    )(_sc_worker)
    return run(em_flat, tags, tr_flat, trt_flat, start, end)


def _finalize_kernel(pv_ref, mv_ref, o_ref):
    pa = pv_ref[0]
    pb = pv_ref[1]
    dot = jnp.sum(pa * pb, axis=1, keepdims=True)
    offa = mv_ref[0, :, 0:1]
    offb = mv_ref[1, :, 0:1]
    sca = mv_ref[0, :, 1:2]
    scb = mv_ref[1, :, 1:2]
    o_ref[...] = (sca + scb) - (jnp.log(dot) + offa + offb)


def kernel(emissions, tags, lengths, transitions, start_transitions, end_transitions):
    del lengths
    trans = transitions[0]
    pv, mv = _sc_crf(emissions.reshape(-1), tags, trans.reshape(-1),
                     trans.T.reshape(-1),
                     start_transitions[0], end_transitions[0])
    out = pl.pallas_call(
        _finalize_kernel,
        out_shape=jax.ShapeDtypeStruct((B, 1), jnp.float32),
    )(pv.reshape(2, B, T), mv.reshape(2, B, T))
    return out[:, 0]


# speedup vs baseline: 63.9504x; 1.3029x over previous
import functools

import jax
import jax.numpy as jnp
from jax import lax
from jax.experimental import pallas as pl
from jax.experimental.pallas import tpu as pltpu
from jax.experimental.pallas import tpu_sc as plsc

B = 16
L = 2048
T = 16
HALF = L // 2
LN2 = 0.6931471805599453

_GDN = lax.GatherDimensionNumbers(
    offset_dims=(), collapsed_slice_dims=(0,), start_index_map=(0,))


def _bcast(vec, j):
    idx = jnp.full((T, 1), j, jnp.int32)
    return lax.gather(vec, idx, _GDN, (1,),
                      mode=lax.GatherScatterMode.PROMISE_IN_BOUNDS)


def _sc_worker(em_hbm, tags_hbm, tr_hbm, st_hbm, en_hbm, pv_out, mv_out,
               em_v, tg_v, tr_v, st_v, en_v, po_v, mo_v):
    b = lax.axis_index("s")
    h = lax.axis_index("c")
    h0 = h == 0
    iota = jnp.arange(T, dtype=jnp.int32)
    hmask = (jnp.zeros((T,), jnp.int32) + h) == 0

    row0 = b * L + h * HALF
    pltpu.sync_copy(em_hbm.at[pl.ds(row0 * T, HALF * T)], em_v)
    tag0 = b * L + (HALF - 8) * h
    pltpu.sync_copy(tags_hbm.at[pl.ds(tag0, 1032)], tg_v.at[pl.ds(0, 1032)])
    pltpu.sync_copy(tr_hbm, tr_v)
    pltpu.sync_copy(st_hbm, st_v)
    pltpu.sync_copy(en_hbm, en_v)

    pair_limit = jnp.where(h0, 1024, 1023)
    hoff = 8 * h

    def score_body(i, acc):
        lanes = iota + i * T
        tga = plsc.load_gather(tg_v, [lanes + hoff]) & 15
        emv = plsc.load_gather(em_v, [lanes * T + tga])
        tgb = plsc.load_gather(tg_v, [lanes + hoff + 1]) & 15
        trv = plsc.load_gather(tr_v, [tga * T + tgb])
        trv = jnp.where(lanes < pair_limit, trv, 0.0)
        return acc + emv + trv

    acc = lax.fori_loop(0, HALF // T, score_body, jnp.zeros((T,), jnp.float32))
    tE = plsc.load_gather(
        tg_v, [jnp.zeros((T,), jnp.int32) + jnp.where(h0, 0, 1031)]) & 15
    sv = plsc.load_gather(st_v, [tE])
    ev = plsc.load_gather(en_v, [tE])
    score = jnp.sum(acc) + jnp.max(jnp.where(hmask, sv, ev))

    mrows = [jnp.exp(tr_v[pl.ds(j * T, T)]) for j in range(T)]

    def matvec(x):
        a0 = _bcast(x, 0) * mrows[0]
        a1 = _bcast(x, 1) * mrows[1]
        a2 = _bcast(x, 2) * mrows[2]
        a3 = _bcast(x, 3) * mrows[3]
        for j in range(4, T, 4):
            a0 = a0 + _bcast(x, j) * mrows[j]
            a1 = a1 + _bcast(x, j + 1) * mrows[j + 1]
            a2 = a2 + _bcast(x, j + 2) * mrows[j + 2]
            a3 = a3 + _bcast(x, j + 3) * mrows[j + 3]
        return (a0 + a1) + (a2 + a3)

    def renorm(pn, off):
        mx = jnp.max(pn)
        ebias = (lax.bitcast_convert_type(mx, jnp.int32) >> 23) & 0xFF
        scale = lax.bitcast_convert_type((254 - ebias) << 23, jnp.float32)
        return pn * scale, off + (ebias - 127).astype(jnp.float32) * LN2

    s_dir = 1 - 2 * h
    c0 = 1 + 1021 * h

    def step(x, i):
        row = i * s_dir + c0
        ee = jnp.exp(em_v[pl.ds(row * T, T)])
        return matvec(x) * ee

    v0 = jnp.where(hmask, st_v[...] + em_v[pl.ds(0, T)],
                   en_v[...] + em_v[pl.ds((HALF - 1) * T, T)])
    m0 = jnp.max(v0)
    p0 = jnp.exp(v0 - m0)

    def body(g, carry):
        p, off = carry
        i = g * 8
        for k in range(8):
            p = step(p, i + k)
        return renorm(p, off)

    p, off = lax.fori_loop(0, 127, body, (p0, m0))
    for k in range(7):
        p = step(p, 1016 + k)
    p = jnp.where(hmask, matvec(p), p)
    p, off = renorm(p, off)

    po_v[...] = p
    mo_v[...] = jnp.where(iota == 0, off, jnp.where(iota == 1, score, 0.0))
    wid = h * B + b
    pltpu.sync_copy(po_v, pv_out.at[pl.ds(wid * T, T)])
    pltpu.sync_copy(mo_v, mv_out.at[pl.ds(wid * T, T)])


def _sc_crf(em_flat, tags, tr_flat, start, end):
    mesh = plsc.VectorSubcoreMesh(core_axis_name="c", subcore_axis_name="s")
    run = functools.partial(
        pl.kernel,
        out_type=(
            jax.ShapeDtypeStruct((2 * B * T,), jnp.float32),
            jax.ShapeDtypeStruct((2 * B * T,), jnp.float32),
        ),
        mesh=mesh,
        compiler_params=pltpu.CompilerParams(needs_layout_passes=False),
        scratch_types=[
            pltpu.VMEM((HALF * T,), jnp.float32),
            pltpu.VMEM((1040,), jnp.int32),
            pltpu.VMEM((T * T,), jnp.float32),
            pltpu.VMEM((T,), jnp.float32),
            pltpu.VMEM((T,), jnp.float32),
            pltpu.VMEM((T,), jnp.float32),
            pltpu.VMEM((T,), jnp.float32),
        ],
    )(_sc_worker)
    return run(em_flat, tags, tr_flat, start, end)


def _finalize_kernel(pv_ref, mv_ref, o_ref):
    pa = pv_ref[0]
    pb = pv_ref[1]
    dot = jnp.sum(pa * pb, axis=1, keepdims=True)
    offa = mv_ref[0, :, 0:1]
    offb = mv_ref[1, :, 0:1]
    sca = mv_ref[0, :, 1:2]
    scb = mv_ref[1, :, 1:2]
    o_ref[...] = (sca + scb) - (jnp.log(dot) + offa + offb)


def kernel(emissions, tags, lengths, transitions, start_transitions, end_transitions):
    del lengths
    trans = transitions[0]
    pv, mv = _sc_crf(emissions.reshape(-1), tags, trans.reshape(-1),
                     start_transitions[0], end_transitions[0])
    out = pl.pallas_call(
        _finalize_kernel,
        out_shape=jax.ShapeDtypeStruct((B, 1), jnp.float32),
    )(pv.reshape(2, B, T), mv.reshape(2, B, T))
    return out[:, 0]


# speedup vs baseline: 65.1433x; 1.0187x over previous
import functools

import jax
import jax.numpy as jnp
from jax import lax
from jax.experimental import pallas as pl
from jax.experimental.pallas import tpu as pltpu
from jax.experimental.pallas import tpu_sc as plsc

B = 16
L = 2048
T = 16
HALF = L // 2
LN2 = 0.6931471805599453

_GDN = lax.GatherDimensionNumbers(
    offset_dims=(), collapsed_slice_dims=(0,), start_index_map=(0,))


def _bcast(vec, j):
    idx = jnp.full((T, 1), j, jnp.int32)
    return lax.gather(vec, idx, _GDN, (1,),
                      mode=lax.GatherScatterMode.PROMISE_IN_BOUNDS)


def _sc_worker(em_hbm, tags_hbm, tr_hbm, st_hbm, en_hbm, pv_out, mv_out,
               em_v, tg_v, tr_v, st_v, en_v, po_v, mo_v):
    b = lax.axis_index("s")
    h = lax.axis_index("c")
    h0 = h == 0
    iota = jnp.arange(T, dtype=jnp.int32)
    hmask = (jnp.zeros((T,), jnp.int32) + h) == 0

    row0 = b * L + h * HALF
    pltpu.sync_copy(
        em_hbm.at[pl.ds(pl.multiple_of(row0 // 8, 8), HALF * T // 128)], em_v)
    tag0 = b * L + (HALF - 8) * h
    pltpu.sync_copy(tags_hbm.at[pl.ds(tag0, 1032)], tg_v.at[pl.ds(0, 1032)])
    pltpu.sync_copy(tr_hbm, tr_v)
    pltpu.sync_copy(st_hbm, st_v)
    pltpu.sync_copy(en_hbm, en_v)

    pair_limit = jnp.where(h0, 1024, 1023)
    hoff = 8 * h

    def score_body(i, acc):
        lanes = iota + i * T
        tga = plsc.load_gather(tg_v, [lanes + hoff]) & 15
        w = lanes * T + tga
        emv = plsc.load_gather(em_v, [w >> 7, w & 127])
        tgb = plsc.load_gather(tg_v, [lanes + hoff + 1]) & 15
        trv = plsc.load_gather(tr_v, [tga * T + tgb])
        trv = jnp.where(lanes < pair_limit, trv, 0.0)
        return acc + emv + trv

    acc = lax.fori_loop(0, HALF // T, score_body, jnp.zeros((T,), jnp.float32))
    tE = plsc.load_gather(
        tg_v, [jnp.zeros((T,), jnp.int32) + jnp.where(h0, 0, 1031)]) & 15
    sv = plsc.load_gather(st_v, [tE])
    ev = plsc.load_gather(en_v, [tE])
    score = jnp.sum(acc) + jnp.max(jnp.where(hmask, sv, ev))

    mrows = [jnp.exp(tr_v[pl.ds(j * T, T)]) for j in range(T)]

    def matvec(x):
        a0 = _bcast(x, 0) * mrows[0]
        a1 = _bcast(x, 1) * mrows[1]
        a2 = _bcast(x, 2) * mrows[2]
        a3 = _bcast(x, 3) * mrows[3]
        for j in range(4, T, 4):
            a0 = a0 + _bcast(x, j) * mrows[j]
            a1 = a1 + _bcast(x, j + 1) * mrows[j + 1]
            a2 = a2 + _bcast(x, j + 2) * mrows[j + 2]
            a3 = a3 + _bcast(x, j + 3) * mrows[j + 3]
        return (a0 + a1) + (a2 + a3)

    def renorm(pn, off):
        mx = jnp.max(pn)
        ebias = (lax.bitcast_convert_type(mx, jnp.int32) >> 23) & 0xFF
        scale = lax.bitcast_convert_type((254 - ebias) << 23, jnp.float32)
        return pn * scale, off + (ebias - 127).astype(jnp.float32) * LN2

    s_dir = 1 - 2 * h
    c0 = 1 + 1021 * h

    def step(x, i):
        row = i * s_dir + c0
        ee = jnp.exp(em_v[row >> 3, pl.ds((row & 7) * T, T)])
        return matvec(x) * ee

    v0 = jnp.where(hmask, st_v[...] + em_v[0, pl.ds(0, T)],
                   en_v[...] + em_v[127, pl.ds(112, T)])
    m0 = jnp.max(v0)
    p0 = jnp.exp(v0 - m0)

    def body(g, carry):
        p, off = carry
        i = g * 8
        for k in range(8):
            p = step(p, i + k)
        return renorm(p, off)

    p, off = lax.fori_loop(0, 127, body, (p0, m0))
    for k in range(7):
        p = step(p, 1016 + k)
    p = jnp.where(hmask, matvec(p), p)
    p, off = renorm(p, off)

    po_v[...] = p
    mo_v[...] = jnp.where(iota == 0, off, jnp.where(iota == 1, score, 0.0))
    wid = h * B + b
    pltpu.sync_copy(po_v, pv_out.at[pl.ds(wid * T, T)])
    pltpu.sync_copy(mo_v, mv_out.at[pl.ds(wid * T, T)])


def _sc_crf(em_flat, tags, tr_flat, start, end):
    mesh = plsc.VectorSubcoreMesh(core_axis_name="c", subcore_axis_name="s")
    run = functools.partial(
        pl.kernel,
        out_type=(
            jax.ShapeDtypeStruct((2 * B * T,), jnp.float32),
            jax.ShapeDtypeStruct((2 * B * T,), jnp.float32),
        ),
        mesh=mesh,
        compiler_params=pltpu.CompilerParams(needs_layout_passes=False),
        scratch_types=[
            pltpu.VMEM((HALF * T // 128, 128), jnp.float32),
            pltpu.VMEM((1040,), jnp.int32),
            pltpu.VMEM((T * T,), jnp.float32),
            pltpu.VMEM((T,), jnp.float32),
            pltpu.VMEM((T,), jnp.float32),
            pltpu.VMEM((T,), jnp.float32),
            pltpu.VMEM((T,), jnp.float32),
        ],
    )(_sc_worker)
    return run(em_flat, tags, tr_flat, start, end)


def _finalize_kernel(pv_ref, mv_ref, o_ref):
    pa = pv_ref[pl.ds(0, B), :]
    pb = pv_ref[pl.ds(B, B), :]
    dot = jnp.sum(pa * pb, axis=1)
    ma = mv_ref[pl.ds(0, B), :]
    mb = mv_ref[pl.ds(B, B), :]
    o_ref[...] = (ma[:, 1] + mb[:, 1]) - (jnp.log(dot) + ma[:, 0] + mb[:, 0])


def kernel(emissions, tags, lengths, transitions, start_transitions, end_transitions):
    del lengths
    trans = transitions[0]
    pv, mv = _sc_crf(emissions.reshape(L * B * T // 128, 128), tags,
                     trans.reshape(-1),
                     start_transitions[0], end_transitions[0])
    return pl.pallas_call(
        _finalize_kernel,
        out_shape=jax.ShapeDtypeStruct((B,), jnp.float32),
    )(pv.reshape(2 * B, T), mv.reshape(2 * B, T))


# speedup vs baseline: 83.3197x; 1.2790x over previous
import functools

import jax
import jax.numpy as jnp
from jax import lax
from jax.experimental import pallas as pl
from jax.experimental.pallas import tpu as pltpu
from jax.experimental.pallas import tpu_sc as plsc

B = 16
L = 2048
T = 16
HALF = L // 2
LN2 = 0.6931471805599453

_GDN = lax.GatherDimensionNumbers(
    offset_dims=(), collapsed_slice_dims=(0,), start_index_map=(0,))


def _bcast(vec, j):
    idx = jnp.full((T, 1), j, jnp.int32)
    return lax.gather(vec, idx, _GDN, (1,),
                      mode=lax.GatherScatterMode.PROMISE_IN_BOUNDS)


def _sc_worker(em_hbm, tags_hbm, tr_hbm, st_hbm, en_hbm, pv_out, mv_out,
               em_v, tg_v, tr_v, st_v, en_v, po_v, mo_v):
    b = lax.axis_index("s")
    h = lax.axis_index("c")
    h0 = h == 0
    iota = jnp.arange(T, dtype=jnp.int32)
    hmask = (jnp.zeros((T,), jnp.int32) + h) == 0

    n0 = b * L + h * HALF
    pltpu.sync_copy(em_hbm.at[:, pl.ds(pl.multiple_of(n0, 128), HALF)], em_v)
    tag0 = b * L + (HALF - 8) * h
    pltpu.sync_copy(tags_hbm.at[pl.ds(tag0, 1032)], tg_v.at[pl.ds(0, 1032)])
    pltpu.sync_copy(tr_hbm.at[0], tr_v)
    pltpu.sync_copy(st_hbm.at[0], st_v)
    pltpu.sync_copy(en_hbm.at[0], en_v)

    pair_limit = jnp.where(h0, 1024, 1023)
    hoff = 8 * h

    def score_body(i, acc):
        lanes = iota + i * T
        tga = plsc.load_gather(tg_v, [lanes + hoff]) & 15
        emv = plsc.load_gather(em_v, [tga, lanes])
        tgb = plsc.load_gather(tg_v, [lanes + hoff + 1]) & 15
        trv = plsc.load_gather(tr_v, [tga, tgb])
        trv = jnp.where(lanes < pair_limit, trv, 0.0)
        return acc + emv + trv

    acc = lax.fori_loop(0, HALF // T, score_body, jnp.zeros((T,), jnp.float32))
    tE = plsc.load_gather(
        tg_v, [jnp.zeros((T,), jnp.int32) + jnp.where(h0, 0, 1031)]) & 15
    sv = plsc.load_gather(st_v, [tE])
    ev = plsc.load_gather(en_v, [tE])
    score = jnp.sum(acc) + jnp.max(jnp.where(hmask, sv, ev))

    mrows = [jnp.exp(tr_v[j, :]) for j in range(T)]

    def matvec(x):
        a0 = _bcast(x, 0) * mrows[0]
        a1 = _bcast(x, 1) * mrows[1]
        a2 = _bcast(x, 2) * mrows[2]
        a3 = _bcast(x, 3) * mrows[3]
        for j in range(4, T, 4):
            a0 = a0 + _bcast(x, j) * mrows[j]
            a1 = a1 + _bcast(x, j + 1) * mrows[j + 1]
            a2 = a2 + _bcast(x, j + 2) * mrows[j + 2]
            a3 = a3 + _bcast(x, j + 3) * mrows[j + 3]
        return (a0 + a1) + (a2 + a3)

    def renorm(pn, off):
        mx = jnp.max(pn)
        ebias = (lax.bitcast_convert_type(mx, jnp.int32) >> 23) & 0xFF
        scale = lax.bitcast_convert_type((254 - ebias) << 23, jnp.float32)
        return pn * scale, off + (ebias - 127).astype(jnp.float32) * LN2

    s_dir = 1 - 2 * h
    c0 = 1 + 1021 * h

    sdv = jnp.zeros((T,), jnp.int32) + s_dir

    def ee_at(rv):
        return jnp.exp(plsc.load_gather(em_v, [iota, rv]))

    ilv = plsc.PackFormat.INTERLEAVED
    mpack = [plsc.pack(mrows[2 * m], mrows[2 * m + 1], format=ilv)
             for m in range(8)]
    evens = ((iota + iota) & 15).reshape(T, 1)
    odds = ((iota + iota + 1) & 15).reshape(T, 1)

    def _gat(vec, idx):
        return lax.gather(vec, idx, _GDN, (1,),
                          mode=lax.GatherScatterMode.PROMISE_IN_BOUNDS)

    def matvecb(pn):
        pe = _gat(pn, evens)
        po = _gat(pn, odds)
        pi = plsc.bitcast(plsc.pack(pe, po, format=ilv), jnp.int32)
        a0 = plsc.bitcast(_bcast(pi, 0), jnp.bfloat16) * mpack[0]
        a1 = plsc.bitcast(_bcast(pi, 1), jnp.bfloat16) * mpack[1]
        a2 = plsc.bitcast(_bcast(pi, 2), jnp.bfloat16) * mpack[2]
        a3 = plsc.bitcast(_bcast(pi, 3), jnp.bfloat16) * mpack[3]
        a0 = a0 + plsc.bitcast(_bcast(pi, 4), jnp.bfloat16) * mpack[4]
        a1 = a1 + plsc.bitcast(_bcast(pi, 5), jnp.bfloat16) * mpack[5]
        a2 = a2 + plsc.bitcast(_bcast(pi, 6), jnp.bfloat16) * mpack[6]
        a3 = a3 + plsc.bitcast(_bcast(pi, 7), jnp.bfloat16) * mpack[7]
        acc32 = (a0 + a1) + (a2 + a3)
        ae, ao = plsc.unpack(acc32, format=ilv)
        return ae + ao

    em0 = plsc.load_gather(em_v, [iota, jnp.zeros((T,), jnp.int32)])
    emz = plsc.load_gather(em_v, [iota, jnp.full((T,), HALF - 1, jnp.int32)])
    v0 = jnp.where(hmask, st_v[...] + em0, en_v[...] + emz)
    m0 = jnp.max(v0)
    p0 = jnp.exp(v0 - m0)

    def body(g, carry):
        p, off = carry
        rv = jnp.zeros((T,), jnp.int32) + ((g * 8) * s_dir + c0)
        ees = []
        for k in range(8):
            ees.append(ee_at(rv))
            rv = rv + sdv
        for k in range(8):
            p = matvecb(p) * ees[k]
        return renorm(p, off)

    p, off = lax.fori_loop(0, 127, body, (p0, m0))
    rv = jnp.zeros((T,), jnp.int32) + (1016 * s_dir + c0)
    ees = []
    for k in range(7):
        ees.append(ee_at(rv))
        rv = rv + sdv
    for k in range(7):
        p = matvecb(p) * ees[k]
    p = jnp.where(hmask, matvec(p), p)
    p, off = renorm(p, off)

    po_v[...] = p
    mo_v[...] = jnp.where(iota == 0, off, jnp.where(iota == 1, score, 0.0))
    wid = h * B + b
    pltpu.sync_copy(po_v, pv_out.at[pl.ds(wid * T, T)])
    pltpu.sync_copy(mo_v, mv_out.at[pl.ds(wid * T, T)])


def _sc_crf(em_flat, tags, tr_flat, start, end):
    mesh = plsc.VectorSubcoreMesh(core_axis_name="c", subcore_axis_name="s")
    run = functools.partial(
        pl.kernel,
        out_type=(
            jax.ShapeDtypeStruct((2 * B * T,), jnp.float32),
            jax.ShapeDtypeStruct((2 * B * T,), jnp.float32),
        ),
        mesh=mesh,
        compiler_params=pltpu.CompilerParams(needs_layout_passes=False),
        scratch_types=[
            pltpu.VMEM((T, HALF), jnp.float32),
            pltpu.VMEM((1040,), jnp.int32),
            pltpu.VMEM((T, T), jnp.float32),
            pltpu.VMEM((T,), jnp.float32),
            pltpu.VMEM((T,), jnp.float32),
            pltpu.VMEM((T,), jnp.float32),
            pltpu.VMEM((T,), jnp.float32),
        ],
    )(_sc_worker)
    return run(em_flat, tags, tr_flat, start, end)


def _finalize_kernel(pv_ref, mv_ref, o_ref):
    lane = jax.lax.iota(jnp.int32, B)
    out = jnp.zeros((B,), jnp.float32)
    for b in range(B):
        pa = pv_ref[pl.ds(b * T, T)]
        pb = pv_ref[pl.ds((B + b) * T, T)]
        dot = jnp.sum(pa * pb)
        ma = mv_ref[pl.ds(b * T, T)]
        mb = mv_ref[pl.ds((B + b) * T, T)]
        comb = (ma[1] + mb[1]) - (jnp.log(dot) + ma[0] + mb[0])
        out = jnp.where(lane == b, comb, out)
    o_ref[...] = out


def kernel(emissions, tags, lengths, transitions, start_transitions, end_transitions):
    del lengths
    pv, mv = _sc_crf(emissions.T, tags, transitions,
                     start_transitions, end_transitions)
    return pl.pallas_call(
        _finalize_kernel,
        out_shape=jax.ShapeDtypeStruct((B,), jnp.float32),
    )(pv, mv)
